# baseline (speedup 1.0000x reference)
import functools

import jax
import jax.numpy as jnp
from jax import lax
from jax.experimental import pallas as pl
from jax.experimental.pallas import tpu as pltpu
from jax.experimental.pallas import tpu_sc as plsc

N_NODES = 10000
N_EDGES = 320000
D_FEAT = 128
HID = 64
EXT = HID + 16

_NC = 2
_NS = 16
_NW = _NC * _NS

_CH = 128
_EPW = N_EDGES // _NW
_NCHUNK = 79
_EPWP = _NCHUNK * _CH
_DPAD = 10100
_NPAD = 10240
_RPT = _NPAD // _NS
_RCOPY = 128

_BLK = 2000
_GRID = N_NODES // _BLK


def _dot_t(a, w):
    return lax.dot_general(a, w, (((1,), (1,)), ((), ())),
                           preferred_element_type=jnp.float32)


def _tc1_body(x_ref, wl_ref, wr_ref, y_ref, z_ref):
    xb = x_ref[...]
    y = _dot_t(xb, wl_ref[...])
    z = _dot_t(xb, wr_ref[...])
    y_ref[...] = jnp.concatenate(
        [y, jnp.ones((y.shape[0], EXT - HID), jnp.float32)], axis=1)
    z_ref[...] = z


def _tc1(x, Wl1, Wr1):
    return pl.pallas_call(
        _tc1_body,
        grid=(_GRID,),
        in_specs=[
            pl.BlockSpec((_BLK, D_FEAT), lambda i: (i, 0)),
            pl.BlockSpec((HID, D_FEAT), lambda i: (0, 0)),
            pl.BlockSpec((HID, D_FEAT), lambda i: (0, 0)),
        ],
        out_specs=[
            pl.BlockSpec((_BLK, EXT), lambda i: (i, 0)),
            pl.BlockSpec((_BLK, HID), lambda i: (i, 0)),
        ],
        out_shape=[
            jax.ShapeDtypeStruct((N_NODES, EXT), jnp.float32),
            jax.ShapeDtypeStruct((N_NODES, HID), jnp.float32),
        ],
    )(x, Wl1, Wr1)


def _tc2_body(aggp_ref, z1_ref, bl1_ref, wl2_ref, wr2_ref,
              y2_ref, z2_ref, recip_ref):
    a = aggp_ref[0] + aggp_ref[1]
    cnt = jnp.max(a[:, HID:EXT], axis=1, keepdims=True)
    recip = 1.0 / jnp.maximum(cnt, 1.0)
    h1 = jnp.maximum(a[:, :HID] * recip + bl1_ref[...] + z1_ref[...], 0.0)
    y2_ref[...] = _dot_t(h1, wl2_ref[...])
    z2_ref[...] = _dot_t(h1, wr2_ref[...])
    recip_ref[...] = recip


def _tc2(aggp, z1, bl1, Wl2, Wr2):
    return pl.pallas_call(
        _tc2_body,
        grid=(_GRID,),
        in_specs=[
            pl.BlockSpec((_NC, _BLK, EXT), lambda i: (0, i, 0)),
            pl.BlockSpec((_BLK, HID), lambda i: (i, 0)),
            pl.BlockSpec((1, HID), lambda i: (0, 0)),
            pl.BlockSpec((HID, HID), lambda i: (0, 0)),
            pl.BlockSpec((HID, HID), lambda i: (0, 0)),
        ],
        out_specs=[
            pl.BlockSpec((_BLK, HID), lambda i: (i, 0)),
            pl.BlockSpec((_BLK, HID), lambda i: (i, 0)),
            pl.BlockSpec((_BLK, 1), lambda i: (i, 0)),
        ],
        out_shape=[
            jax.ShapeDtypeStruct((N_NODES, HID), jnp.float32),
            jax.ShapeDtypeStruct((N_NODES, HID), jnp.float32),
            jax.ShapeDtypeStruct((N_NODES, 1), jnp.float32),
        ],
    )(aggp, z1, bl1, Wl2, Wr2)


def _tc3_body(aggp_ref, z2_ref, bl2_ref, recip_ref,
              wf1_ref, bf1_ref, wf2_ref, bf2_ref, out_ref, acc_ref):
    i = pl.program_id(0)

    @pl.when(i == 0)
    def _():
        acc_ref[...] = jnp.zeros_like(acc_ref)

    a = aggp_ref[0] + aggp_ref[1]
    h2 = jnp.maximum(a * recip_ref[...] + bl2_ref[...] + z2_ref[...], 0.0)
    acc_ref[...] += jnp.sum(h2, axis=0, keepdims=True)

    @pl.when(i == pl.num_programs(0) - 1)
    def _():
        g = acc_ref[...] * (1.0 / N_NODES)
        h = jnp.maximum(_dot_t(g, wf1_ref[...]) + bf1_ref[...], 0.0)
        o = _dot_t(h, wf2_ref[...]) + bf2_ref[...]
        m = jnp.max(o, axis=1, keepdims=True)
        e = jnp.exp(o - m)
        out_ref[...] = e / jnp.sum(e, axis=1, keepdims=True)


def _tc3(aggp, z2, bl2, recip, Wf1, bf1, Wf2, bf2):
    nout = Wf2.shape[0]
    return pl.pallas_call(
        _tc3_body,
        grid=(_GRID,),
        in_specs=[
            pl.BlockSpec((_NC, _BLK, HID), lambda i: (0, i, 0)),
            pl.BlockSpec((_BLK, HID), lambda i: (i, 0)),
            pl.BlockSpec((1, HID), lambda i: (0, 0)),
            pl.BlockSpec((_BLK, 1), lambda i: (i, 0)),
            pl.BlockSpec((HID, HID), lambda i: (0, 0)),
            pl.BlockSpec((1, HID), lambda i: (0, 0)),
            pl.BlockSpec((nout, HID), lambda i: (0, 0)),
            pl.BlockSpec((1, nout), lambda i: (0, 0)),
        ],
        out_specs=pl.BlockSpec((1, nout), lambda i: (0, 0)),
        out_shape=jax.ShapeDtypeStruct((1, nout), jnp.float32),
        scratch_shapes=[pltpu.VMEM((1, HID), jnp.float32)],
    )(aggp, z2, bl2, recip, Wf1, bf1, Wf2, bf2)


def _sc_agg(y, src3d, dst3d, d, stage_y):
    mesh = plsc.VectorSubcoreMesh(core_axis_name="c", subcore_axis_name="s",
                                  num_cores=_NC, num_subcores=_NS)

    @functools.partial(
        pl.kernel,
        out_type=jax.ShapeDtypeStruct((_NC, _NPAD, d), jnp.float32),
        mesh=mesh,
        scratch_types=[
            pltpu.VMEM((_NCHUNK, _CH), jnp.int32),
            pltpu.VMEM((_NCHUNK, _CH), jnp.int32),
            pltpu.VMEM((_CH, d), jnp.float32),
            pltpu.VMEM((_CH, d), jnp.float32),
            pltpu.VMEM((_RCOPY, d), jnp.float32),
            pltpu.VMEM_SHARED((N_NODES if stage_y else 1, d), jnp.float32),
            pltpu.VMEM_SHARED((_NPAD, d), jnp.float32),
            pltpu.SemaphoreType.DMA,
            pltpu.SemaphoreType.DMA,
        ],
        compiler_params=pltpu.CompilerParams(use_tc_tiling_on_sc=False),
    )
    def k(y_hbm, src_hbm, dst_hbm, out_hbm, src_v, dst_v, rows0_v, rows1_v,
          buf_v, y_sh, agg_sh, sem0, sem1):
        c = lax.axis_index("c")
        s = lax.axis_index("s")
        wid = s * _NC + c

        if stage_y:
            @pl.when(s == 0)
            def _():
                pltpu.async_copy(y_hbm, y_sh, sem1)

        def zrow(i, _):
            for jj in range(d // 16):
                buf_v[i, pl.ds(jj * 16, 16)] = jnp.zeros((16,), jnp.float32)
            return 0
        lax.fori_loop(0, _RCOPY, zrow, 0)

        for r in range(_RPT // _RCOPY):
            pltpu.sync_copy(
                buf_v, agg_sh.at[pl.ds(s * _RPT + r * _RCOPY, _RCOPY)])

        pltpu.sync_copy(src_hbm.at[wid], src_v)
        pltpu.sync_copy(dst_hbm.at[wid], dst_v)

        if stage_y:
            @pl.when(s == 0)
            def _():
                pltpu.make_async_copy(y_hbm, y_sh, sem1).wait()

        plsc.subcore_barrier()

        y_src = y_sh if stage_y else y_hbm

        def gather(j, buf, sem):
            return pltpu.async_copy(y_src.at[src_v.at[j]], buf, sem)

        def gwait(j, buf, sem):
            pltpu.make_async_copy(y_src.at[src_v.at[j]], buf, sem).wait()

        def scat(j, buf):
            pltpu.sync_copy(buf, agg_sh.at[dst_v.at[j]], add=True)

        gather(0, rows0_v, sem0)

        def body(i, _):
            a = 2 * i
            gwait(a, rows0_v, sem0)
            gather(a + 1, rows1_v, sem1)
            scat(a, rows0_v)
            gwait(a + 1, rows1_v, sem1)
            gather(a + 2, rows0_v, sem0)
            scat(a + 1, rows1_v)
            return 0
        lax.fori_loop(0, (_NCHUNK - 1) // 2, body, 0)

        gwait(_NCHUNK - 1, rows0_v, sem0)
        scat(_NCHUNK - 1, rows0_v)

        plsc.subcore_barrier()

        for r in range(_RPT // _RCOPY):
            base = s * _RPT + r * _RCOPY
            pltpu.sync_copy(agg_sh.at[pl.ds(base, _RCOPY)], buf_v)
            pltpu.sync_copy(buf_v, out_hbm.at[c, pl.ds(base, _RCOPY)])

    return k(y, src3d, dst3d)


def kernel(x, edge_index, Wl1, bl1, Wr1, Wl2, bl2, Wr2, Wf1, bf1, Wf2, bf2):
    pad = ((0, 0), (0, _EPWP - _EPW))
    src3d = jnp.pad(edge_index[0].reshape(_NW, _EPW), pad,
                    constant_values=0).reshape(_NW, _NCHUNK, _CH)
    dst3d = jnp.pad(edge_index[1].reshape(_NW, _EPW), pad,
                    constant_values=_DPAD).reshape(_NW, _NCHUNK, _CH)

    y1, z1 = _tc1(x, Wl1, Wr1)
    aggp1 = _sc_agg(y1, src3d, dst3d, EXT, stage_y=False)
    y2, z2, recip = _tc2(aggp1, z1, bl1.reshape(1, HID), Wl2, Wr2)
    aggp2 = _sc_agg(y2, src3d, dst3d, HID, stage_y=True)
    return _tc3(aggp2, z2, bl2.reshape(1, HID), recip,
                Wf1, bf1.reshape(1, HID), Wf2, bf2.reshape(1, Wf2.shape[0]))


# speedup vs baseline: 1.3243x; 1.3243x over previous
import functools

import jax
import jax.numpy as jnp
from jax import lax
from jax.experimental import pallas as pl
from jax.experimental.pallas import tpu as pltpu
from jax.experimental.pallas import tpu_sc as plsc

N_NODES = 10000
N_EDGES = 320000
D_FEAT = 128
HID = 64
EXT = HID + 16

_NC = 2
_NS = 16
_NW = _NC * _NS

_CH = 80
_EPW = N_EDGES // _NW
_NCHUNK = _EPW // _CH
_NPAD = 10240
_RPT = _NPAD // _NS
_RCOPY = _CH

_BLK = 2000
_GRID = N_NODES // _BLK


def _dot_t(a, w):
    return lax.dot_general(a, w, (((1,), (1,)), ((), ())),
                           preferred_element_type=jnp.float32)


def _tc1_body(x_ref, wl_ref, wr_ref, y_ref, z_ref):
    xb = x_ref[...]
    y = _dot_t(xb, wl_ref[...])
    z = _dot_t(xb, wr_ref[...])
    y_ref[...] = jnp.concatenate(
        [y, jnp.ones((y.shape[0], EXT - HID), jnp.float32)], axis=1)
    z_ref[...] = z


def _tc1(x, Wl1, Wr1):
    return pl.pallas_call(
        _tc1_body,
        grid=(_GRID,),
        in_specs=[
            pl.BlockSpec((_BLK, D_FEAT), lambda i: (i, 0)),
            pl.BlockSpec((HID, D_FEAT), lambda i: (0, 0)),
            pl.BlockSpec((HID, D_FEAT), lambda i: (0, 0)),
        ],
        out_specs=[
            pl.BlockSpec((_BLK, EXT), lambda i: (i, 0)),
            pl.BlockSpec((_BLK, HID), lambda i: (i, 0)),
        ],
        out_shape=[
            jax.ShapeDtypeStruct((N_NODES, EXT), jnp.float32),
            jax.ShapeDtypeStruct((N_NODES, HID), jnp.float32),
        ],
    )(x, Wl1, Wr1)


def _tc2_body(aggp_ref, z1_ref, bl1_ref, wl2_ref, wr2_ref,
              y2_ref, z2_ref, recip_ref):
    a = aggp_ref[0] + aggp_ref[1]
    cnt = jnp.max(a[:, HID:EXT], axis=1, keepdims=True)
    recip = 1.0 / jnp.maximum(cnt, 1.0)
    h1 = jnp.maximum(a[:, :HID] * recip + bl1_ref[...] + z1_ref[...], 0.0)
    y2_ref[...] = _dot_t(h1, wl2_ref[...])
    z2_ref[...] = _dot_t(h1, wr2_ref[...])
    recip_ref[...] = recip


def _tc2(aggp, z1, bl1, Wl2, Wr2):
    return pl.pallas_call(
        _tc2_body,
        grid=(_GRID,),
        in_specs=[
            pl.BlockSpec((_NC, _BLK, EXT), lambda i: (0, i, 0)),
            pl.BlockSpec((_BLK, HID), lambda i: (i, 0)),
            pl.BlockSpec((1, HID), lambda i: (0, 0)),
            pl.BlockSpec((HID, HID), lambda i: (0, 0)),
            pl.BlockSpec((HID, HID), lambda i: (0, 0)),
        ],
        out_specs=[
            pl.BlockSpec((_BLK, HID), lambda i: (i, 0)),
            pl.BlockSpec((_BLK, HID), lambda i: (i, 0)),
            pl.BlockSpec((_BLK, 1), lambda i: (i, 0)),
        ],
        out_shape=[
            jax.ShapeDtypeStruct((N_NODES, HID), jnp.float32),
            jax.ShapeDtypeStruct((N_NODES, HID), jnp.float32),
            jax.ShapeDtypeStruct((N_NODES, 1), jnp.float32),
        ],
    )(aggp, z1, bl1, Wl2, Wr2)


def _tc3_body(aggp_ref, z2_ref, bl2_ref, recip_ref,
              wf1_ref, bf1_ref, wf2_ref, bf2_ref, out_ref, acc_ref):
    i = pl.program_id(0)

    @pl.when(i == 0)
    def _():
        acc_ref[...] = jnp.zeros_like(acc_ref)

    a = aggp_ref[0] + aggp_ref[1]
    h2 = jnp.maximum(a * recip_ref[...] + bl2_ref[...] + z2_ref[...], 0.0)
    acc_ref[...] += jnp.sum(h2, axis=0, keepdims=True)

    @pl.when(i == pl.num_programs(0) - 1)
    def _():
        g = acc_ref[...] * (1.0 / N_NODES)
        h = jnp.maximum(_dot_t(g, wf1_ref[...]) + bf1_ref[...], 0.0)
        o = _dot_t(h, wf2_ref[...]) + bf2_ref[...]
        m = jnp.max(o, axis=1, keepdims=True)
        e = jnp.exp(o - m)
        out_ref[...] = e / jnp.sum(e, axis=1, keepdims=True)


def _tc3(aggp, z2, bl2, recip, Wf1, bf1, Wf2, bf2):
    nout = Wf2.shape[0]
    return pl.pallas_call(
        _tc3_body,
        grid=(_GRID,),
        in_specs=[
            pl.BlockSpec((_NC, _BLK, HID), lambda i: (0, i, 0)),
            pl.BlockSpec((_BLK, HID), lambda i: (i, 0)),
            pl.BlockSpec((1, HID), lambda i: (0, 0)),
            pl.BlockSpec((_BLK, 1), lambda i: (i, 0)),
            pl.BlockSpec((HID, HID), lambda i: (0, 0)),
            pl.BlockSpec((1, HID), lambda i: (0, 0)),
            pl.BlockSpec((nout, HID), lambda i: (0, 0)),
            pl.BlockSpec((1, nout), lambda i: (0, 0)),
        ],
        out_specs=pl.BlockSpec((1, nout), lambda i: (0, 0)),
        out_shape=jax.ShapeDtypeStruct((1, nout), jnp.float32),
        scratch_shapes=[pltpu.VMEM((1, HID), jnp.float32)],
    )(aggp, z2, bl2, recip, Wf1, bf1, Wf2, bf2)


def _sc_agg(y, edges3d, d, stage_y):
    mesh = plsc.VectorSubcoreMesh(core_axis_name="c", subcore_axis_name="s",
                                  num_cores=_NC, num_subcores=_NS)

    @functools.partial(
        pl.kernel,
        out_type=jax.ShapeDtypeStruct((_NC, _NPAD, d), jnp.float32),
        mesh=mesh,
        scratch_types=[
            pltpu.VMEM((_NCHUNK, _CH), jnp.int32),
            pltpu.VMEM((_CH,), jnp.int32),
            pltpu.VMEM((_CH,), jnp.int32),
            pltpu.VMEM((_CH,), jnp.int32),
            pltpu.VMEM((_CH,), jnp.int32),
            pltpu.VMEM((_CH, d), jnp.float32),
            pltpu.VMEM((_CH, d), jnp.float32),
            pltpu.VMEM_SHARED((N_NODES if stage_y else 1, d), jnp.float32),
            pltpu.VMEM_SHARED((_NPAD, d), jnp.float32),
            pltpu.SemaphoreType.DMA,
            pltpu.SemaphoreType.DMA,
        ],
        compiler_params=pltpu.CompilerParams(use_tc_tiling_on_sc=False),
    )
    def k(y_hbm, edges_hbm, out_hbm, pk_v, src0_v, src1_v, dst0_v, dst1_v,
          rows0_v, rows1_v, y_sh, agg_sh, sem0, sem1):
        c = lax.axis_index("c")
        s = lax.axis_index("s")
        wid = s * _NC + c

        if stage_y:
            @pl.when(s == 0)
            def _():
                pltpu.async_copy(y_hbm, y_sh, sem1)

        pltpu.async_copy(edges_hbm.at[wid], pk_v, sem0)

        def zrow(i, _):
            for jj in range(d // 16):
                rows0_v[i, pl.ds(jj * 16, 16)] = jnp.zeros((16,), jnp.float32)
            return 0
        lax.fori_loop(0, _RCOPY, zrow, 0)
        for r in range(_RPT // _RCOPY):
            pltpu.sync_copy(
                rows0_v, agg_sh.at[pl.ds(s * _RPT + r * _RCOPY, _RCOPY)])

        pltpu.make_async_copy(edges_hbm.at[wid], pk_v, sem0).wait()
        if stage_y:
            @pl.when(s == 0)
            def _():
                pltpu.make_async_copy(y_hbm, y_sh, sem1).wait()

        plsc.subcore_barrier()

        y_src = y_sh if stage_y else y_hbm

        def unpack(j, sbuf, dbuf):
            for jj in range(_CH // 16):
                p = pk_v[j, pl.ds(jj * 16, 16)]
                sbuf[pl.ds(jj * 16, 16)] = lax.shift_right_logical(p, 14)
                dbuf[pl.ds(jj * 16, 16)] = lax.bitwise_and(p, 16383)

        def gather(sbuf, buf, sem):
            return pltpu.async_copy(y_src.at[sbuf], buf, sem)

        def gwait(sbuf, buf, sem):
            pltpu.make_async_copy(y_src.at[sbuf], buf, sem).wait()

        def scat(dbuf, buf):
            pltpu.sync_copy(buf, agg_sh.at[dbuf], add=True)

        unpack(0, src0_v, dst0_v)
        gather(src0_v, rows0_v, sem0)

        def body(i, _):
            a = 2 * i
            unpack(a + 1, src1_v, dst1_v)
            gwait(src0_v, rows0_v, sem0)
            gather(src1_v, rows1_v, sem1)
            scat(dst0_v, rows0_v)
            unpack(a + 2, src0_v, dst0_v)
            gwait(src1_v, rows1_v, sem1)
            gather(src0_v, rows0_v, sem0)
            scat(dst1_v, rows1_v)
            return 0
        lax.fori_loop(0, (_NCHUNK - 1) // 2, body, 0)

        gwait(src0_v, rows0_v, sem0)
        scat(dst0_v, rows0_v)

        plsc.subcore_barrier()

        for r in range(_RPT // _RCOPY):
            base = s * _RPT + r * _RCOPY
            pltpu.sync_copy(agg_sh.at[pl.ds(base, _RCOPY)], rows0_v)
            pltpu.sync_copy(rows0_v, out_hbm.at[c, pl.ds(base, _RCOPY)])

    return k(y, edges3d)


def kernel(x, edge_index, Wl1, bl1, Wr1, Wl2, bl2, Wr2, Wf1, bf1, Wf2, bf2):
    edges3d = (jnp.left_shift(edge_index[0], 14) | edge_index[1]).reshape(
        _NW, _NCHUNK, _CH)

    y1, z1 = _tc1(x, Wl1, Wr1)
    aggp1 = _sc_agg(y1, edges3d, EXT, stage_y=True)
    y2, z2, recip = _tc2(aggp1, z1, bl1.reshape(1, HID), Wl2, Wr2)
    aggp2 = _sc_agg(y2, edges3d, HID, stage_y=True)
    return _tc3(aggp2, z2, bl2.reshape(1, HID), recip,
                Wf1, bf1.reshape(1, HID), Wf2, bf2.reshape(1, Wf2.shape[0]))


# speedup vs baseline: 1.3396x; 1.0115x over previous
import functools

import jax
import jax.numpy as jnp
from jax import lax
from jax.experimental import pallas as pl
from jax.experimental.pallas import tpu as pltpu
from jax.experimental.pallas import tpu_sc as plsc

N_NODES = 10000
N_EDGES = 320000
D_FEAT = 128
HID = 64
EXT = HID + 16

_NC = 2
_NS = 16
_NW = _NC * _NS

_CH = 80
_EPW = N_EDGES // _NW
_NCHUNK = _EPW // _CH
_NPAD = 10240
_RPT = _NPAD // _NS
_RCOPY = _CH

_BLK = 2000
_GRID = N_NODES // _BLK


def _dot_t(a, w):
    return lax.dot_general(a, w, (((1,), (1,)), ((), ())),
                           preferred_element_type=jnp.float32)


def _tc1_body(x_ref, wl_ref, wr_ref, y_ref, z_ref):
    xb = x_ref[...]
    y = _dot_t(xb, wl_ref[...])
    z = _dot_t(xb, wr_ref[...])
    y_ref[...] = jnp.concatenate(
        [y, jnp.ones((y.shape[0], EXT - HID), jnp.float32)], axis=1)
    z_ref[...] = z


def _tc1(x, Wl1, Wr1):
    return pl.pallas_call(
        _tc1_body,
        out_shape=[
            jax.ShapeDtypeStruct((N_NODES, EXT), jnp.float32),
            jax.ShapeDtypeStruct((N_NODES, HID), jnp.float32),
        ],
    )(x, Wl1, Wr1)


def _tc2_body(aggp_ref, z1_ref, bl1_ref, wl2_ref, wr2_ref, y2_ref, z2_ref):
    a = aggp_ref[0, :N_NODES, :] + aggp_ref[1, :N_NODES, :]
    cnt = jnp.max(a[:, HID:EXT], axis=1, keepdims=True)
    recip = 1.0 / jnp.maximum(cnt, 1.0)
    h1 = jnp.maximum(a[:, :HID] * recip + bl1_ref[...] + z1_ref[...], 0.0)
    y2_ref[...] = _dot_t(h1, wl2_ref[...])
    z2_ref[...] = jnp.concatenate(
        [_dot_t(h1, wr2_ref[...]),
         jnp.broadcast_to(recip, (N_NODES, EXT - HID))], axis=1)


def _tc2(aggp, z1, bl1, Wl2, Wr2):
    return pl.pallas_call(
        _tc2_body,
        out_shape=[
            jax.ShapeDtypeStruct((N_NODES, HID), jnp.float32),
            jax.ShapeDtypeStruct((N_NODES, EXT), jnp.float32),
        ],
    )(aggp, z1, bl1, Wl2, Wr2)


def _tc3_body(aggp_ref, z2_ref, bl2_ref, wf1_ref, bf1_ref, wf2_ref, bf2_ref,
              out_ref):
    a = aggp_ref[0, :N_NODES, :] + aggp_ref[1, :N_NODES, :]
    recip = z2_ref[:, HID:HID + 1]
    h2 = jnp.maximum(a * recip + bl2_ref[...] + z2_ref[:, :HID], 0.0)
    g = jnp.sum(h2, axis=0, keepdims=True) * (1.0 / N_NODES)
    h = jnp.maximum(_dot_t(g, wf1_ref[...]) + bf1_ref[...], 0.0)
    o = _dot_t(h, wf2_ref[...]) + bf2_ref[...]
    m = jnp.max(o, axis=1, keepdims=True)
    e = jnp.exp(o - m)
    out_ref[...] = e / jnp.sum(e, axis=1, keepdims=True)


def _tc3(aggp, z2, bl2, Wf1, bf1, Wf2, bf2):
    nout = Wf2.shape[0]
    return pl.pallas_call(
        _tc3_body,
        out_shape=jax.ShapeDtypeStruct((1, nout), jnp.float32),
    )(aggp, z2, bl2, Wf1, bf1, Wf2, bf2)


def _sc_agg(y, edges3d, d, stage_y):
    mesh = plsc.VectorSubcoreMesh(core_axis_name="c", subcore_axis_name="s",
                                  num_cores=_NC, num_subcores=_NS)

    @functools.partial(
        pl.kernel,
        out_type=jax.ShapeDtypeStruct((_NC, _NPAD, d), jnp.float32),
        mesh=mesh,
        scratch_types=[
            pltpu.VMEM((_NCHUNK, _CH), jnp.int32),
            pltpu.VMEM((_CH,), jnp.int32),
            pltpu.VMEM((_CH,), jnp.int32),
            pltpu.VMEM((_CH,), jnp.int32),
            pltpu.VMEM((_CH,), jnp.int32),
            pltpu.VMEM((_CH, d), jnp.float32),
            pltpu.VMEM((_CH, d), jnp.float32),
            pltpu.VMEM_SHARED((N_NODES if stage_y else 1, d), jnp.float32),
            pltpu.VMEM_SHARED((_NPAD, d), jnp.float32),
            pltpu.SemaphoreType.DMA,
            pltpu.SemaphoreType.DMA,
        ],
        compiler_params=pltpu.CompilerParams(use_tc_tiling_on_sc=False),
    )
    def k(y_hbm, edges_hbm, out_hbm, pk_v, src0_v, src1_v, dst0_v, dst1_v,
          rows0_v, rows1_v, y_sh, agg_sh, sem0, sem1):
        c = lax.axis_index("c")
        s = lax.axis_index("s")
        wid = s * _NC + c

        if stage_y:
            @pl.when(s == 0)
            def _():
                pltpu.async_copy(y_hbm, y_sh, sem1)

        pltpu.async_copy(edges_hbm.at[wid], pk_v, sem0)

        def zrow(i, _):
            for jj in range(d // 16):
                rows0_v[i, pl.ds(jj * 16, 16)] = jnp.zeros((16,), jnp.float32)
            return 0
        lax.fori_loop(0, _RCOPY, zrow, 0)
        for r in range(_RPT // _RCOPY):
            pltpu.sync_copy(
                rows0_v, agg_sh.at[pl.ds(s * _RPT + r * _RCOPY, _RCOPY)])

        pltpu.make_async_copy(edges_hbm.at[wid], pk_v, sem0).wait()
        if stage_y:
            @pl.when(s == 0)
            def _():
                pltpu.make_async_copy(y_hbm, y_sh, sem1).wait()

        plsc.subcore_barrier()

        y_src = y_sh if stage_y else y_hbm

        def unpack(j, sbuf, dbuf):
            for jj in range(_CH // 16):
                p = pk_v[j, pl.ds(jj * 16, 16)]
                sbuf[pl.ds(jj * 16, 16)] = lax.shift_right_logical(p, 14)
                dbuf[pl.ds(jj * 16, 16)] = lax.bitwise_and(p, 16383)

        def gather(sbuf, buf, sem):
            return pltpu.async_copy(y_src.at[sbuf], buf, sem)

        def gwait(sbuf, buf, sem):
            pltpu.make_async_copy(y_src.at[sbuf], buf, sem).wait()

        def scat(dbuf, buf):
            pltpu.sync_copy(buf, agg_sh.at[dbuf], add=True)

        unpack(0, src0_v, dst0_v)
        gather(src0_v, rows0_v, sem0)

        def body(i, _):
            a = 2 * i
            unpack(a + 1, src1_v, dst1_v)
            gwait(src0_v, rows0_v, sem0)
            gather(src1_v, rows1_v, sem1)
            scat(dst0_v, rows0_v)
            unpack(a + 2, src0_v, dst0_v)
            gwait(src1_v, rows1_v, sem1)
            gather(src0_v, rows0_v, sem0)
            scat(dst1_v, rows1_v)
            return 0
        lax.fori_loop(0, (_NCHUNK - 1) // 2, body, 0)

        gwait(src0_v, rows0_v, sem0)
        scat(dst0_v, rows0_v)

        plsc.subcore_barrier()

        for r in range(_RPT // _RCOPY):
            base = s * _RPT + r * _RCOPY
            pltpu.sync_copy(agg_sh.at[pl.ds(base, _RCOPY)], rows0_v)
            pltpu.sync_copy(rows0_v, out_hbm.at[c, pl.ds(base, _RCOPY)])

    return k(y, edges3d)


def kernel(x, edge_index, Wl1, bl1, Wr1, Wl2, bl2, Wr2, Wf1, bf1, Wf2, bf2):
    edges3d = (jnp.left_shift(edge_index[0], 14) | edge_index[1]).reshape(
        _NW, _NCHUNK, _CH)

    y1, z1 = _tc1(x, Wl1, Wr1)
    aggp1 = _sc_agg(y1, edges3d, EXT, stage_y=True)
    y2, z2 = _tc2(aggp1, z1, bl1.reshape(1, HID), Wl2, Wr2)
    aggp2 = _sc_agg(y2, edges3d, HID, stage_y=True)
    return _tc3(aggp2, z2, bl2.reshape(1, HID),
                Wf1, bf1.reshape(1, HID), Wf2, bf2.reshape(1, Wf2.shape[0]))


# speedup vs baseline: 1.5525x; 1.1589x over previous
import functools

import jax
import jax.numpy as jnp
from jax import lax
from jax.experimental import pallas as pl
from jax.experimental.pallas import tpu as pltpu
from jax.experimental.pallas import tpu_sc as plsc

N_NODES = 10000
N_EDGES = 320000
D_FEAT = 128
HID = 64
EXT = HID + 16

_NC = 2
_NS = 16
_NW = _NC * _NS

_CH = 80
_EPW = N_EDGES // _NW
_NCHUNK = _EPW // _CH
_NPAD = 10240
_RPT = _NPAD // _NS
_RCOPY = _CH

_BLK = 2000
_GRID = N_NODES // _BLK


def _dot_t(a, w):
    return lax.dot_general(a, w, (((1,), (1,)), ((), ())),
                           preferred_element_type=jnp.float32)


def _tc1_body(x_ref, wl_ref, wr_ref, y_ref, z_ref):
    xb = x_ref[...]
    y = _dot_t(xb, wl_ref[...])
    z = _dot_t(xb, wr_ref[...])
    y_ref[...] = jnp.concatenate(
        [y, jnp.ones((y.shape[0], EXT - HID), jnp.float32)], axis=1)
    z_ref[...] = z


def _tc1(x, Wl1, Wr1):
    return pl.pallas_call(
        _tc1_body,
        out_shape=[
            jax.ShapeDtypeStruct((N_NODES, EXT), jnp.float32),
            jax.ShapeDtypeStruct((N_NODES, HID), jnp.float32),
        ],
    )(x, Wl1, Wr1)


def _tc2_body(aggp_ref, z1_ref, bl1_ref, wl2_ref, wr2_ref, y2_ref, z2_ref):
    a = aggp_ref[0, :N_NODES, :] + aggp_ref[1, :N_NODES, :]
    cnt = jnp.max(a[:, HID:EXT], axis=1, keepdims=True)
    recip = 1.0 / jnp.maximum(cnt, 1.0)
    h1 = jnp.maximum(a[:, :HID] * recip + bl1_ref[...] + z1_ref[...], 0.0)
    y2_ref[...] = _dot_t(h1, wl2_ref[...]).astype(jnp.bfloat16)
    z2_ref[...] = jnp.concatenate(
        [_dot_t(h1, wr2_ref[...]),
         jnp.broadcast_to(recip, (N_NODES, EXT - HID))], axis=1)


def _tc2(aggp, z1, bl1, Wl2, Wr2):
    return pl.pallas_call(
        _tc2_body,
        out_shape=[
            jax.ShapeDtypeStruct((N_NODES, HID), jnp.bfloat16),
            jax.ShapeDtypeStruct((N_NODES, EXT), jnp.float32),
        ],
    )(aggp, z1, bl1, Wl2, Wr2)


def _tc3_body(aggp_ref, z2_ref, bl2_ref, wf1_ref, bf1_ref, wf2_ref, bf2_ref,
              out_ref):
    a = (aggp_ref[0, :N_NODES, :] + aggp_ref[1, :N_NODES, :]).astype(
        jnp.float32)
    recip = z2_ref[:, HID:HID + 1]
    h2 = jnp.maximum(a * recip + bl2_ref[...] + z2_ref[:, :HID], 0.0)
    g = jnp.sum(h2, axis=0, keepdims=True) * (1.0 / N_NODES)
    h = jnp.maximum(_dot_t(g, wf1_ref[...]) + bf1_ref[...], 0.0)
    o = _dot_t(h, wf2_ref[...]) + bf2_ref[...]
    m = jnp.max(o, axis=1, keepdims=True)
    e = jnp.exp(o - m)
    out_ref[...] = e / jnp.sum(e, axis=1, keepdims=True)


def _tc3(aggp, z2, bl2, Wf1, bf1, Wf2, bf2):
    nout = Wf2.shape[0]
    return pl.pallas_call(
        _tc3_body,
        out_shape=jax.ShapeDtypeStruct((1, nout), jnp.float32),
    )(aggp, z2, bl2, Wf1, bf1, Wf2, bf2)


def _sc_agg(y, edges3d, d, stage_y):
    dt = y.dtype
    lw = 32 if dt == jnp.bfloat16 else 16
    mesh = plsc.VectorSubcoreMesh(core_axis_name="c", subcore_axis_name="s",
                                  num_cores=_NC, num_subcores=_NS)

    @functools.partial(
        pl.kernel,
        out_type=jax.ShapeDtypeStruct((_NC, _NPAD, d), dt),
        mesh=mesh,
        scratch_types=[
            pltpu.VMEM((_NCHUNK, _CH), jnp.int32),
            pltpu.VMEM((_CH,), jnp.int32),
            pltpu.VMEM((_CH,), jnp.int32),
            pltpu.VMEM((_CH,), jnp.int32),
            pltpu.VMEM((_CH,), jnp.int32),
            pltpu.VMEM((_CH, d), dt),
            pltpu.VMEM((_CH, d), dt),
            pltpu.VMEM_SHARED((N_NODES if stage_y else 1, d), dt),
            pltpu.VMEM_SHARED((_NPAD, d), dt),
            pltpu.SemaphoreType.DMA,
            pltpu.SemaphoreType.DMA,
        ],
        compiler_params=pltpu.CompilerParams(use_tc_tiling_on_sc=False),
    )
    def k(y_hbm, edges_hbm, out_hbm, pk_v, src0_v, src1_v, dst0_v, dst1_v,
          rows0_v, rows1_v, y_sh, agg_sh, sem0, sem1):
        c = lax.axis_index("c")
        s = lax.axis_index("s")
        wid = s * _NC + c

        if stage_y:
            @pl.when(s == 0)
            def _():
                pltpu.async_copy(y_hbm, y_sh, sem1)

        pltpu.async_copy(edges_hbm.at[wid], pk_v, sem0)

        def zrow(i, _):
            for jj in range(d // lw):
                rows0_v[i, pl.ds(jj * lw, lw)] = jnp.zeros((lw,), dt)
            return 0
        lax.fori_loop(0, _RCOPY, zrow, 0)
        for r in range(_RPT // _RCOPY):
            pltpu.sync_copy(
                rows0_v, agg_sh.at[pl.ds(s * _RPT + r * _RCOPY, _RCOPY)])

        pltpu.make_async_copy(edges_hbm.at[wid], pk_v, sem0).wait()
        if stage_y:
            @pl.when(s == 0)
            def _():
                pltpu.make_async_copy(y_hbm, y_sh, sem1).wait()

        plsc.subcore_barrier()

        y_src = y_sh if stage_y else y_hbm

        def unpack(j, sbuf, dbuf):
            for jj in range(_CH // 16):
                p = pk_v[j, pl.ds(jj * 16, 16)]
                sbuf[pl.ds(jj * 16, 16)] = lax.shift_right_logical(p, 14)
                dbuf[pl.ds(jj * 16, 16)] = lax.bitwise_and(p, 16383)

        def gather(sbuf, buf, sem):
            return pltpu.async_copy(y_src.at[sbuf], buf, sem)

        def gwait(sbuf, buf, sem):
            pltpu.make_async_copy(y_src.at[sbuf], buf, sem).wait()

        def scat(dbuf, buf):
            pltpu.sync_copy(buf, agg_sh.at[dbuf], add=True)

        unpack(0, src0_v, dst0_v)
        gather(src0_v, rows0_v, sem0)

        def body(i, _):
            a = 2 * i
            unpack(a + 1, src1_v, dst1_v)
            gwait(src0_v, rows0_v, sem0)
            gather(src1_v, rows1_v, sem1)
            scat(dst0_v, rows0_v)
            unpack(a + 2, src0_v, dst0_v)
            gwait(src1_v, rows1_v, sem1)
            gather(src0_v, rows0_v, sem0)
            scat(dst1_v, rows1_v)
            return 0
        lax.fori_loop(0, (_NCHUNK - 1) // 2, body, 0)

        gwait(src0_v, rows0_v, sem0)
        scat(dst0_v, rows0_v)

        plsc.subcore_barrier()

        for r in range(_RPT // _RCOPY):
            base = s * _RPT + r * _RCOPY
            pltpu.sync_copy(agg_sh.at[pl.ds(base, _RCOPY)], rows0_v)
            pltpu.sync_copy(rows0_v, out_hbm.at[c, pl.ds(base, _RCOPY)])

    return k(y, edges3d)


def kernel(x, edge_index, Wl1, bl1, Wr1, Wl2, bl2, Wr2, Wf1, bf1, Wf2, bf2):
    edges3d = (jnp.left_shift(edge_index[0], 14) | edge_index[1]).reshape(
        _NW, _NCHUNK, _CH)

    y1, z1 = _tc1(x, Wl1, Wr1)
    aggp1 = _sc_agg(y1, edges3d, EXT, stage_y=True)
    y2, z2 = _tc2(aggp1, z1, bl1.reshape(1, HID), Wl2, Wr2)
    aggp2 = _sc_agg(y2, edges3d, HID, stage_y=True)
    return _tc3(aggp2, z2, bl2.reshape(1, HID),
                Wf1, bf1.reshape(1, HID), Wf2, bf2.reshape(1, Wf2.shape[0]))


# speedup vs baseline: 1.8066x; 1.1637x over previous
import functools

import jax
import jax.numpy as jnp
from jax import lax
from jax.experimental import pallas as pl
from jax.experimental.pallas import tpu as pltpu
from jax.experimental.pallas import tpu_sc as plsc

N_NODES = 10000
N_EDGES = 320000
D_FEAT = 128
HID = 64
EXT = HID + 16

_NC = 2
_NS = 16
_NW = _NC * _NS

_CH = 80
_EPW = N_EDGES // _NW
_NCHUNK = _EPW // _CH
_NPAD = 10240
_RPT = _NPAD // _NS
_RCOPY = _CH

_BLK = 2000
_GRID = N_NODES // _BLK


def _dot_t(a, w):
    return lax.dot_general(a, w, (((1,), (1,)), ((), ())),
                           preferred_element_type=jnp.float32)


def _tc1_body(x_ref, wl_ref, wr_ref, y_ref, z_ref):
    xb = x_ref[...]
    y_ref[...] = _dot_t(xb, wl_ref[...]).astype(jnp.bfloat16)
    z_ref[...] = _dot_t(xb, wr_ref[...])


def _tc1(x, Wl1, Wr1):
    return pl.pallas_call(
        _tc1_body,
        out_shape=[
            jax.ShapeDtypeStruct((N_NODES, HID), jnp.bfloat16),
            jax.ShapeDtypeStruct((N_NODES, HID), jnp.float32),
        ],
    )(x, Wl1, Wr1)


def _tc2_body(aggp_ref, cntp_ref, z1_ref, bl1_ref, wl2_ref, wr2_ref,
              y2_ref, z2_ref):
    a = (aggp_ref[0, :N_NODES, :] + aggp_ref[1, :N_NODES, :]).astype(
        jnp.float32)
    cv = cntp_ref[0, :N_NODES, :] + cntp_ref[1, :N_NODES, :]
    cnt = jnp.max(cv, axis=1, keepdims=True)
    recip = 1.0 / jnp.maximum(cnt, 1.0)
    h1 = jnp.maximum(a * recip + bl1_ref[...] + z1_ref[...], 0.0)
    y2_ref[...] = _dot_t(h1, wl2_ref[...]).astype(jnp.bfloat16)
    z2_ref[...] = jnp.concatenate(
        [_dot_t(h1, wr2_ref[...]),
         jnp.broadcast_to(recip, (N_NODES, EXT - HID))], axis=1)


def _tc2(aggp, cntp, z1, bl1, Wl2, Wr2):
    return pl.pallas_call(
        _tc2_body,
        out_shape=[
            jax.ShapeDtypeStruct((N_NODES, HID), jnp.bfloat16),
            jax.ShapeDtypeStruct((N_NODES, EXT), jnp.float32),
        ],
    )(aggp, cntp, z1, bl1, Wl2, Wr2)


def _tc3_body(aggp_ref, z2_ref, bl2_ref, wf1_ref, bf1_ref, wf2_ref, bf2_ref,
              out_ref):
    a = (aggp_ref[0, :N_NODES, :] + aggp_ref[1, :N_NODES, :]).astype(
        jnp.float32)
    recip = z2_ref[:, HID:HID + 1]
    h2 = jnp.maximum(a * recip + bl2_ref[...] + z2_ref[:, :HID], 0.0)
    g = jnp.sum(h2, axis=0, keepdims=True) * (1.0 / N_NODES)
    h = jnp.maximum(_dot_t(g, wf1_ref[...]) + bf1_ref[...], 0.0)
    o = _dot_t(h, wf2_ref[...]) + bf2_ref[...]
    m = jnp.max(o, axis=1, keepdims=True)
    e = jnp.exp(o - m)
    out_ref[...] = e / jnp.sum(e, axis=1, keepdims=True)


def _tc3(aggp, z2, bl2, Wf1, bf1, Wf2, bf2):
    nout = Wf2.shape[0]
    return pl.pallas_call(
        _tc3_body,
        out_shape=jax.ShapeDtypeStruct((1, nout), jnp.float32),
    )(aggp, z2, bl2, Wf1, bf1, Wf2, bf2)


_CNTW = 16


def _sc_agg(y, edges3d, d, stage_y, with_cnt=False):
    dt = y.dtype
    lw = 32 if dt == jnp.bfloat16 else 16
    mesh = plsc.VectorSubcoreMesh(core_axis_name="c", subcore_axis_name="s",
                                  num_cores=_NC, num_subcores=_NS)

    out_type = [jax.ShapeDtypeStruct((_NC, _NPAD, d), dt)]
    if with_cnt:
        out_type.append(
            jax.ShapeDtypeStruct((_NC, _NPAD, _CNTW), jnp.float32))

    @functools.partial(
        pl.kernel,
        out_type=out_type,
        mesh=mesh,
        scratch_types=[
            pltpu.VMEM((_NCHUNK, _CH), jnp.int32),
            pltpu.VMEM((_CH,), jnp.int32),
            pltpu.VMEM((_CH,), jnp.int32),
            pltpu.VMEM((_CH,), jnp.int32),
            pltpu.VMEM((_CH,), jnp.int32),
            pltpu.VMEM((_CH, d), dt),
            pltpu.VMEM((_CH, d), dt),
            pltpu.VMEM((_CH, _CNTW), jnp.float32),
            pltpu.VMEM_SHARED((N_NODES if stage_y else 1, d), dt),
            pltpu.VMEM_SHARED((_NPAD, d), dt),
            pltpu.VMEM_SHARED((_NPAD if with_cnt else 1, _CNTW), jnp.float32),
            pltpu.SemaphoreType.DMA,
            pltpu.SemaphoreType.DMA,
        ],
        compiler_params=pltpu.CompilerParams(use_tc_tiling_on_sc=False),
    )
    def k(y_hbm, edges_hbm, *rest):
        if with_cnt:
            (out_hbm, cnt_hbm, pk_v, src0_v, src1_v, dst0_v, dst1_v,
             rows0_v, rows1_v, ones_v, y_sh, agg_sh, cnt_sh, sem0, sem1) = rest
        else:
            (out_hbm, pk_v, src0_v, src1_v, dst0_v, dst1_v,
             rows0_v, rows1_v, ones_v, y_sh, agg_sh, cnt_sh, sem0, sem1) = rest
        c = lax.axis_index("c")
        s = lax.axis_index("s")
        wid = s * _NC + c

        if stage_y:
            @pl.when(s == 0)
            def _():
                pltpu.async_copy(y_hbm, y_sh, sem1)

        pltpu.async_copy(edges_hbm.at[wid], pk_v, sem0)

        def zrow(i, _):
            for jj in range(d // lw):
                rows0_v[i, pl.ds(jj * lw, lw)] = jnp.zeros((lw,), dt)
            return 0
        lax.fori_loop(0, _RCOPY, zrow, 0)
        for r in range(_RPT // _RCOPY):
            pltpu.sync_copy(
                rows0_v, agg_sh.at[pl.ds(s * _RPT + r * _RCOPY, _RCOPY)])

        if with_cnt:
            def czrow(i, val):
                ones_v[i, pl.ds(0, 16)] = jnp.full((16,), val, jnp.float32)
                return val
            lax.fori_loop(0, _CH, lambda i, v: czrow(i, 0.0), 0.0)
            for r in range(_RPT // _RCOPY):
                pltpu.sync_copy(
                    ones_v,
                    cnt_sh.at[pl.ds(s * _RPT + r * _RCOPY, _RCOPY)])
            lax.fori_loop(0, _CH, lambda i, v: czrow(i, 1.0), 1.0)

        pltpu.make_async_copy(edges_hbm.at[wid], pk_v, sem0).wait()
        if stage_y:
            @pl.when(s == 0)
            def _():
                pltpu.make_async_copy(y_hbm, y_sh, sem1).wait()

        plsc.subcore_barrier()

        y_src = y_sh if stage_y else y_hbm

        def unpack(j, sbuf, dbuf):
            for jj in range(_CH // 16):
                p = pk_v[j, pl.ds(jj * 16, 16)]
                sbuf[pl.ds(jj * 16, 16)] = lax.shift_right_logical(p, 14)
                dbuf[pl.ds(jj * 16, 16)] = lax.bitwise_and(p, 16383)

        def gather(sbuf, buf, sem):
            return pltpu.async_copy(y_src.at[sbuf], buf, sem)

        def gwait(sbuf, buf, sem):
            pltpu.make_async_copy(y_src.at[sbuf], buf, sem).wait()

        def scat(dbuf, buf):
            pltpu.sync_copy(buf, agg_sh.at[dbuf], add=True)
            if with_cnt:
                pltpu.sync_copy(ones_v, cnt_sh.at[dbuf], add=True)

        unpack(0, src0_v, dst0_v)
        gather(src0_v, rows0_v, sem0)

        def body(i, _):
            a = 2 * i
            unpack(a + 1, src1_v, dst1_v)
            gwait(src0_v, rows0_v, sem0)
            gather(src1_v, rows1_v, sem1)
            scat(dst0_v, rows0_v)
            unpack(a + 2, src0_v, dst0_v)
            gwait(src1_v, rows1_v, sem1)
            gather(src0_v, rows0_v, sem0)
            scat(dst1_v, rows1_v)
            return 0
        lax.fori_loop(0, (_NCHUNK - 1) // 2, body, 0)

        gwait(src0_v, rows0_v, sem0)
        scat(dst0_v, rows0_v)

        plsc.subcore_barrier()

        for r in range(_RPT // _RCOPY):
            base = s * _RPT + r * _RCOPY
            pltpu.sync_copy(agg_sh.at[pl.ds(base, _RCOPY)], rows0_v)
            pltpu.sync_copy(rows0_v, out_hbm.at[c, pl.ds(base, _RCOPY)])
            if with_cnt:
                pltpu.sync_copy(cnt_sh.at[pl.ds(base, _RCOPY)], ones_v)
                pltpu.sync_copy(ones_v, cnt_hbm.at[c, pl.ds(base, _RCOPY)])

    res = k(y, edges3d)
    return res if with_cnt else res[0]


def kernel(x, edge_index, Wl1, bl1, Wr1, Wl2, bl2, Wr2, Wf1, bf1, Wf2, bf2):
    edges3d = (jnp.left_shift(edge_index[0], 14) | edge_index[1]).reshape(
        _NW, _NCHUNK, _CH)

    y1, z1 = _tc1(x, Wl1, Wr1)
    aggp1, cntp1 = _sc_agg(y1, edges3d, HID, stage_y=True, with_cnt=True)
    y2, z2 = _tc2(aggp1, cntp1, z1, bl1.reshape(1, HID), Wl2, Wr2)
    aggp2 = _sc_agg(y2, edges3d, HID, stage_y=True)
    return _tc3(aggp2, z2, bl2.reshape(1, HID),
                Wf1, bf1.reshape(1, HID), Wf2, bf2.reshape(1, Wf2.shape[0]))


# speedup vs baseline: 1.9556x; 1.0825x over previous
import functools

import jax
import jax.numpy as jnp
from jax import lax
from jax.experimental import pallas as pl
from jax.experimental.pallas import tpu as pltpu
from jax.experimental.pallas import tpu_sc as plsc

N_NODES = 10000
N_EDGES = 320000
D_FEAT = 128
HID = 64
EXT = HID + 16

_NC = 2
_NS = 16
_NW = _NC * _NS

_CH = 80
_EPW = N_EDGES // _NW
_NCHUNK = _EPW // _CH
_NPAD = 10240
_RPT = _NPAD // _NS
_RCOPY = _CH

_BLK = 2000
_GRID = N_NODES // _BLK


def _dot_t(a, w):
    return lax.dot_general(a, w, (((1,), (1,)), ((), ())),
                           preferred_element_type=jnp.float32)


def _tc1_body(x_ref, wl_ref, wr_ref, e_ref, y_ref, z_ref, pk_ref):
    xb = x_ref[...]
    y_ref[...] = _dot_t(xb, wl_ref[...]).astype(jnp.bfloat16)
    z_ref[...] = _dot_t(xb, wr_ref[...])
    pk_ref[...] = jnp.left_shift(e_ref[0], 14) | e_ref[1]


def _tc1(x, Wl1, Wr1, edge_index):
    return pl.pallas_call(
        _tc1_body,
        out_shape=[
            jax.ShapeDtypeStruct((N_NODES, HID), jnp.bfloat16),
            jax.ShapeDtypeStruct((N_NODES, HID), jnp.float32),
            jax.ShapeDtypeStruct((N_EDGES,), jnp.int32),
        ],
    )(x, Wl1, Wr1, edge_index)


def _tc2_body(aggp_ref, cntp_ref, z1_ref, bl1_ref, wl2_ref, wr2_ref,
              y2_ref, z2_ref):
    a = (aggp_ref[0, :N_NODES, :] + aggp_ref[1, :N_NODES, :]).astype(
        jnp.float32)
    cv = cntp_ref[0, :N_NODES, :] + cntp_ref[1, :N_NODES, :]
    cnt = jnp.max(cv, axis=1, keepdims=True)
    recip = 1.0 / jnp.maximum(cnt, 1.0)
    h1 = jnp.maximum(a * recip + bl1_ref[...] + z1_ref[...], 0.0)
    y2_ref[...] = _dot_t(h1, wl2_ref[...]).astype(jnp.bfloat16)
    z2_ref[...] = jnp.concatenate(
        [_dot_t(h1, wr2_ref[...]),
         jnp.broadcast_to(recip, (N_NODES, EXT - HID))], axis=1)


def _tc2(aggp, cntp, z1, bl1, Wl2, Wr2):
    return pl.pallas_call(
        _tc2_body,
        out_shape=[
            jax.ShapeDtypeStruct((N_NODES, HID), jnp.bfloat16),
            jax.ShapeDtypeStruct((N_NODES, EXT), jnp.float32),
        ],
    )(aggp, cntp, z1, bl1, Wl2, Wr2)


def _tc3_body(aggp_ref, z2_ref, bl2_ref, wf1_ref, bf1_ref, wf2_ref, bf2_ref,
              out_ref):
    a = (aggp_ref[0, :N_NODES, :] + aggp_ref[1, :N_NODES, :]).astype(
        jnp.float32)
    recip = z2_ref[:, HID:HID + 1]
    h2 = jnp.maximum(a * recip + bl2_ref[...] + z2_ref[:, :HID], 0.0)
    g = jnp.sum(h2, axis=0, keepdims=True) * (1.0 / N_NODES)
    h = jnp.maximum(_dot_t(g, wf1_ref[...]) + bf1_ref[...], 0.0)
    o = _dot_t(h, wf2_ref[...]) + bf2_ref[...]
    m = jnp.max(o, axis=1, keepdims=True)
    e = jnp.exp(o - m)
    out_ref[...] = e / jnp.sum(e, axis=1, keepdims=True)


def _tc3(aggp, z2, bl2, Wf1, bf1, Wf2, bf2):
    nout = Wf2.shape[0]
    return pl.pallas_call(
        _tc3_body,
        out_shape=jax.ShapeDtypeStruct((1, nout), jnp.float32),
    )(aggp, z2, bl2, Wf1, bf1, Wf2, bf2)


_CNTW = 16


def _sc_agg(y, edges3d, d, stage_y, with_cnt=False):
    dt = y.dtype
    lw = 32 if dt == jnp.bfloat16 else 16
    mesh = plsc.VectorSubcoreMesh(core_axis_name="c", subcore_axis_name="s",
                                  num_cores=_NC, num_subcores=_NS)

    out_type = [jax.ShapeDtypeStruct((_NC, _NPAD, d), dt)]
    if with_cnt:
        out_type.append(
            jax.ShapeDtypeStruct((_NC, _NPAD, _CNTW), jnp.float32))

    @functools.partial(
        pl.kernel,
        out_type=out_type,
        mesh=mesh,
        scratch_types=[
            pltpu.VMEM((_NCHUNK, _CH), jnp.int32),
            pltpu.VMEM((_CH,), jnp.int32),
            pltpu.VMEM((_CH,), jnp.int32),
            pltpu.VMEM((_CH,), jnp.int32),
            pltpu.VMEM((_CH,), jnp.int32),
            pltpu.VMEM((_CH, d), dt),
            pltpu.VMEM((_CH, d), dt),
            pltpu.VMEM((_CH, _CNTW), jnp.float32),
            pltpu.VMEM_SHARED((N_NODES if stage_y else 1, d), dt),
            pltpu.VMEM_SHARED((_NPAD, d), dt),
            pltpu.VMEM_SHARED((_NPAD if with_cnt else 1, _CNTW), jnp.float32),
            pltpu.SemaphoreType.DMA,
            pltpu.SemaphoreType.DMA,
        ],
        compiler_params=pltpu.CompilerParams(use_tc_tiling_on_sc=False),
    )
    def k(y_hbm, edges_hbm, *rest):
        if with_cnt:
            (out_hbm, cnt_hbm, pk_v, src0_v, src1_v, dst0_v, dst1_v,
             rows0_v, rows1_v, ones_v, y_sh, agg_sh, cnt_sh, sem0, sem1) = rest
        else:
            (out_hbm, pk_v, src0_v, src1_v, dst0_v, dst1_v,
             rows0_v, rows1_v, ones_v, y_sh, agg_sh, cnt_sh, sem0, sem1) = rest
        c = lax.axis_index("c")
        s = lax.axis_index("s")
        wid = s * _NC + c

        if stage_y:
            @pl.when(s == 0)
            def _():
                pltpu.async_copy(y_hbm, y_sh, sem1)

        pltpu.async_copy(edges_hbm.at[wid], pk_v, sem0)

        def zrow(i, _):
            for jj in range(d // lw):
                rows0_v[i, pl.ds(jj * lw, lw)] = jnp.zeros((lw,), dt)
            return 0
        lax.fori_loop(0, _RCOPY, zrow, 0)
        for r in range(_RPT // _RCOPY):
            pltpu.sync_copy(
                rows0_v, agg_sh.at[pl.ds(s * _RPT + r * _RCOPY, _RCOPY)])

        if with_cnt:
            def czrow(i, val):
                ones_v[i, pl.ds(0, 16)] = jnp.full((16,), val, jnp.float32)
                return val
            lax.fori_loop(0, _CH, lambda i, v: czrow(i, 0.0), 0.0)
            for r in range(_RPT // _RCOPY):
                pltpu.sync_copy(
                    ones_v,
                    cnt_sh.at[pl.ds(s * _RPT + r * _RCOPY, _RCOPY)])
            lax.fori_loop(0, _CH, lambda i, v: czrow(i, 1.0), 1.0)

        pltpu.make_async_copy(edges_hbm.at[wid], pk_v, sem0).wait()
        if stage_y:
            @pl.when(s == 0)
            def _():
                pltpu.make_async_copy(y_hbm, y_sh, sem1).wait()

        plsc.subcore_barrier()

        y_src = y_sh if stage_y else y_hbm

        def unpack(j, sbuf, dbuf):
            for jj in range(_CH // 16):
                p = pk_v[j, pl.ds(jj * 16, 16)]
                sbuf[pl.ds(jj * 16, 16)] = lax.shift_right_logical(p, 14)
                dbuf[pl.ds(jj * 16, 16)] = lax.bitwise_and(p, 16383)

        def gather(sbuf, buf, sem):
            return pltpu.async_copy(y_src.at[sbuf], buf, sem)

        def gwait(sbuf, buf, sem):
            pltpu.make_async_copy(y_src.at[sbuf], buf, sem).wait()

        def scat(dbuf, buf):
            pltpu.sync_copy(buf, agg_sh.at[dbuf], add=True)
            if with_cnt:
                pltpu.sync_copy(ones_v, cnt_sh.at[dbuf], add=True)

        unpack(0, src0_v, dst0_v)
        gather(src0_v, rows0_v, sem0)

        def body(i, _):
            a = 2 * i
            unpack(a + 1, src1_v, dst1_v)
            gwait(src0_v, rows0_v, sem0)
            gather(src1_v, rows1_v, sem1)
            scat(dst0_v, rows0_v)
            unpack(a + 2, src0_v, dst0_v)
            gwait(src1_v, rows1_v, sem1)
            gather(src0_v, rows0_v, sem0)
            scat(dst1_v, rows1_v)
            return 0
        lax.fori_loop(0, (_NCHUNK - 1) // 2, body, 0)

        gwait(src0_v, rows0_v, sem0)
        scat(dst0_v, rows0_v)

        plsc.subcore_barrier()

        for r in range(_RPT // _RCOPY):
            base = s * _RPT + r * _RCOPY
            pltpu.sync_copy(agg_sh.at[pl.ds(base, _RCOPY)], rows0_v)
            pltpu.sync_copy(rows0_v, out_hbm.at[c, pl.ds(base, _RCOPY)])
            if with_cnt:
                pltpu.sync_copy(cnt_sh.at[pl.ds(base, _RCOPY)], ones_v)
                pltpu.sync_copy(ones_v, cnt_hbm.at[c, pl.ds(base, _RCOPY)])

    res = k(y, edges3d)
    return res if with_cnt else res[0]


def kernel(x, edge_index, Wl1, bl1, Wr1, Wl2, bl2, Wr2, Wf1, bf1, Wf2, bf2):
    y1, z1, packed = _tc1(x, Wl1, Wr1, edge_index)
    edges3d = packed.reshape(_NW, _NCHUNK, _CH)
    aggp1, cntp1 = _sc_agg(y1, edges3d, HID, stage_y=True, with_cnt=True)
    y2, z2 = _tc2(aggp1, cntp1, z1, bl1.reshape(1, HID), Wl2, Wr2)
    aggp2 = _sc_agg(y2, edges3d, HID, stage_y=True)
    return _tc3(aggp2, z2, bl2.reshape(1, HID),
                Wf1, bf1.reshape(1, HID), Wf2, bf2.reshape(1, Wf2.shape[0]))


# speedup vs baseline: 1.9646x; 1.0046x over previous
import functools

import jax
import jax.numpy as jnp
from jax import lax
from jax.experimental import pallas as pl
from jax.experimental.pallas import tpu as pltpu
from jax.experimental.pallas import tpu_sc as plsc

N_NODES = 10000
N_EDGES = 320000
D_FEAT = 128
HID = 64
EXT = HID + 16

_NC = 2
_NS = 16
_NW = _NC * _NS

_CH = 80
_EPW = N_EDGES // _NW
_NCHUNK = _EPW // _CH
_NPAD = 10240
_RPT = _NPAD // _NS
_RCOPY = _CH

_BLK = 2000
_GRID = N_NODES // _BLK


def _dot_t(a, w):
    return lax.dot_general(a, w, (((1,), (1,)), ((), ())),
                           preferred_element_type=jnp.float32)


def _tc1_body(x_ref, wl_ref, wr_ref, e_ref, y_ref, z_ref, pk_ref):
    xb = x_ref[...]
    y_ref[...] = _dot_t(xb, wl_ref[...]).astype(jnp.bfloat16)
    z_ref[...] = _dot_t(xb, wr_ref[...])
    pk_ref[...] = jnp.left_shift(e_ref[0], 14) | e_ref[1]


def _tc1(x, Wl1, Wr1, edge_index):
    return pl.pallas_call(
        _tc1_body,
        out_shape=[
            jax.ShapeDtypeStruct((N_NODES, HID), jnp.bfloat16),
            jax.ShapeDtypeStruct((N_NODES, HID), jnp.float32),
            jax.ShapeDtypeStruct((N_EDGES,), jnp.int32),
        ],
    )(x, Wl1, Wr1, edge_index)


def _tc2_body(aggp_ref, cntp_ref, z1_ref, bl1_ref, wl2_ref, wr2_ref,
              y2_ref, z2_ref):
    a = (aggp_ref[0, :N_NODES, :] + aggp_ref[1, :N_NODES, :]).astype(
        jnp.float32)
    cv = cntp_ref[0, :N_NODES, :] + cntp_ref[1, :N_NODES, :]
    cnt = jnp.max(cv, axis=1, keepdims=True)
    recip = 1.0 / jnp.maximum(cnt, 1.0)
    h1 = jnp.maximum(a * recip + bl1_ref[...] + z1_ref[...], 0.0)
    y2_ref[...] = _dot_t(h1, wl2_ref[...]).astype(jnp.bfloat16)
    z2_ref[...] = jnp.concatenate(
        [_dot_t(h1, wr2_ref[...]),
         jnp.broadcast_to(recip, (N_NODES, EXT - HID))], axis=1)


def _tc2(aggp, cntp, z1, bl1, Wl2, Wr2):
    return pl.pallas_call(
        _tc2_body,
        out_shape=[
            jax.ShapeDtypeStruct((N_NODES, HID), jnp.bfloat16),
            jax.ShapeDtypeStruct((N_NODES, EXT), jnp.float32),
        ],
    )(aggp, cntp, z1, bl1, Wl2, Wr2)


def _tc3_body(aggp_ref, z2_ref, bl2_ref, wf1_ref, bf1_ref, wf2_ref, bf2_ref,
              out_ref):
    a = (aggp_ref[0, :N_NODES, :] + aggp_ref[1, :N_NODES, :]).astype(
        jnp.float32)
    recip = z2_ref[:, HID:HID + 1]
    h2 = jnp.maximum(a * recip + bl2_ref[...] + z2_ref[:, :HID], 0.0)
    g = jnp.sum(h2, axis=0, keepdims=True) * (1.0 / N_NODES)
    h = jnp.maximum(_dot_t(g, wf1_ref[...]) + bf1_ref[...], 0.0)
    o = _dot_t(h, wf2_ref[...]) + bf2_ref[...]
    m = jnp.max(o, axis=1, keepdims=True)
    e = jnp.exp(o - m)
    out_ref[...] = e / jnp.sum(e, axis=1, keepdims=True)


def _tc3(aggp, z2, bl2, Wf1, bf1, Wf2, bf2):
    nout = Wf2.shape[0]
    return pl.pallas_call(
        _tc3_body,
        out_shape=jax.ShapeDtypeStruct((1, nout), jnp.float32),
    )(aggp, z2, bl2, Wf1, bf1, Wf2, bf2)


_CNTW = 16


def _sc_agg(y, edges3d, d, stage_y, with_cnt=False):
    dt = y.dtype
    lw = 32 if dt == jnp.bfloat16 else 16
    mesh = plsc.VectorSubcoreMesh(core_axis_name="c", subcore_axis_name="s",
                                  num_cores=_NC, num_subcores=_NS)

    out_type = [jax.ShapeDtypeStruct((_NC, _NPAD, d), dt)]
    if with_cnt:
        out_type.append(
            jax.ShapeDtypeStruct((_NC, _NPAD, _CNTW), jnp.float32))

    @functools.partial(
        pl.kernel,
        out_type=out_type,
        mesh=mesh,
        scratch_types=[
            pltpu.VMEM((_NCHUNK, _CH), jnp.int32),
            [pltpu.VMEM((_CH,), jnp.int32)] * 3,
            [pltpu.VMEM((_CH,), jnp.int32)] * 3,
            [pltpu.VMEM((_CH, d), dt)] * 3,
            pltpu.VMEM((_CH, _CNTW), jnp.float32),
            pltpu.VMEM_SHARED((N_NODES if stage_y else 1, d), dt),
            pltpu.VMEM_SHARED((_NPAD, d), dt),
            pltpu.VMEM_SHARED((_NPAD if with_cnt else 1, _CNTW), jnp.float32),
            [pltpu.SemaphoreType.DMA] * 3,
            [pltpu.SemaphoreType.DMA] * 3,
        ],
        compiler_params=pltpu.CompilerParams(use_tc_tiling_on_sc=False),
    )
    def k(y_hbm, edges_hbm, *rest):
        if with_cnt:
            (out_hbm, cnt_hbm, pk_v, src_v, dst_v, rows_v, ones_v,
             y_sh, agg_sh, cnt_sh, gsem, ssem) = rest
        else:
            (out_hbm, pk_v, src_v, dst_v, rows_v, ones_v,
             y_sh, agg_sh, cnt_sh, gsem, ssem) = rest
        rows0_v = rows_v[0]
        c = lax.axis_index("c")
        s = lax.axis_index("s")
        wid = s * _NC + c

        if stage_y:
            @pl.when(s == 0)
            def _():
                pltpu.async_copy(y_hbm, y_sh, gsem[1])

        pltpu.async_copy(edges_hbm.at[wid], pk_v, gsem[0])

        def zrow(i, _):
            for jj in range(d // lw):
                rows0_v[i, pl.ds(jj * lw, lw)] = jnp.zeros((lw,), dt)
            return 0
        lax.fori_loop(0, _RCOPY, zrow, 0)
        for r in range(_RPT // _RCOPY):
            pltpu.sync_copy(
                rows0_v, agg_sh.at[pl.ds(s * _RPT + r * _RCOPY, _RCOPY)])

        if with_cnt:
            def czrow(i, val):
                ones_v[i, pl.ds(0, 16)] = jnp.full((16,), val, jnp.float32)
                return val
            lax.fori_loop(0, _CH, lambda i, v: czrow(i, 0.0), 0.0)
            for r in range(_RPT // _RCOPY):
                pltpu.sync_copy(
                    ones_v,
                    cnt_sh.at[pl.ds(s * _RPT + r * _RCOPY, _RCOPY)])
            lax.fori_loop(0, _CH, lambda i, v: czrow(i, 1.0), 1.0)

        pltpu.make_async_copy(edges_hbm.at[wid], pk_v, gsem[0]).wait()
        if stage_y:
            @pl.when(s == 0)
            def _():
                pltpu.make_async_copy(y_hbm, y_sh, gsem[1]).wait()

        plsc.subcore_barrier()

        y_src = y_sh if stage_y else y_hbm

        def ups(j, b):
            for jj in range(_CH // 16):
                p = pk_v[j, pl.ds(jj * 16, 16)]
                src_v[b][pl.ds(jj * 16, 16)] = lax.shift_right_logical(p, 14)

        def upd(j, b):
            for jj in range(_CH // 16):
                p = pk_v[j, pl.ds(jj * 16, 16)]
                dst_v[b][pl.ds(jj * 16, 16)] = lax.bitwise_and(p, 16383)

        def gi(j, b):
            pltpu.async_copy(y_src.at[src_v[b]], rows_v[b], gsem[b])

        def gw(b):
            pltpu.make_async_copy(
                y_src.at[src_v[b]], rows_v[b], gsem[b]).wait()

        def si(b):
            pltpu.async_copy(rows_v[b], agg_sh.at[dst_v[b]], ssem[b],
                             add=True)
            if with_cnt:
                pltpu.sync_copy(ones_v, cnt_sh.at[dst_v[b]], add=True)

        def sw(b):
            pltpu.make_async_copy(
                rows_v[b], agg_sh.at[dst_v[b]], ssem[b]).wait()

        ups(0, 0); upd(0, 0); gi(0, 0)
        gw(0); ups(1, 1); upd(1, 1); gi(1, 1); si(0)
        gw(1); ups(2, 2); upd(2, 2); gi(2, 2); si(1)

        def body(i, _):
            j = 3 * i + 2
            for k in range(3):
                b = (2 + k) % 3
                bn = (3 + k) % 3
                gw(b)
                sw(bn)
                ups(j + k + 1, bn)
                upd(j + k + 1, bn)
                gi(j + k + 1, bn)
                si(b)
            return 0
        lax.fori_loop(0, (_NCHUNK - 5) // 3, body, 0)

        gw(2); sw(0); ups(_NCHUNK - 2, 0); upd(_NCHUNK - 2, 0)
        gi(_NCHUNK - 2, 0); si(2)
        gw(0); sw(1); ups(_NCHUNK - 1, 1); upd(_NCHUNK - 1, 1)
        gi(_NCHUNK - 1, 1); si(0)
        gw(1); si(1)
        sw(2); sw(0); sw(1)

        plsc.subcore_barrier()

        for r in range(_RPT // _RCOPY):
            base = s * _RPT + r * _RCOPY
            pltpu.sync_copy(agg_sh.at[pl.ds(base, _RCOPY)], rows0_v)
            pltpu.sync_copy(rows0_v, out_hbm.at[c, pl.ds(base, _RCOPY)])
            if with_cnt:
                pltpu.sync_copy(cnt_sh.at[pl.ds(base, _RCOPY)], ones_v)
                pltpu.sync_copy(ones_v, cnt_hbm.at[c, pl.ds(base, _RCOPY)])

    res = k(y, edges3d)
    return res if with_cnt else res[0]


def kernel(x, edge_index, Wl1, bl1, Wr1, Wl2, bl2, Wr2, Wf1, bf1, Wf2, bf2):
    y1, z1, packed = _tc1(x, Wl1, Wr1, edge_index)
    edges3d = packed.reshape(_NW, _NCHUNK, _CH)
    aggp1, cntp1 = _sc_agg(y1, edges3d, HID, stage_y=True, with_cnt=True)
    y2, z2 = _tc2(aggp1, cntp1, z1, bl1.reshape(1, HID), Wl2, Wr2)
    aggp2 = _sc_agg(y2, edges3d, HID, stage_y=True)
    return _tc3(aggp2, z2, bl2.reshape(1, HID),
                Wf1, bf1.reshape(1, HID), Wf2, bf2.reshape(1, Wf2.shape[0]))
